# baseline (speedup 1.0000x reference)
import functools

import jax
import jax.numpy as jnp
from jax.experimental import pallas as pl
from jax.experimental.pallas import tpu as pltpu

_EPS = 1e-5


def _fused_encoder(x_ref, m_ref, wpre_ref, gpre_ref, bpre_ref,
                   w1_ref, g1_ref, b1_ref,
                   w2_ref, g2_ref, b2_ref,
                   wout_ref, bout_ref, out_ref, *, n_pts, tile):
    inv = 1.0 / jnp.sqrt(1.0 + _EPS)
    bf = jnp.bfloat16
    x3 = x_ref[...].astype(bf)
    m3 = m_ref[...].astype(bf)
    wpre = (wpre_ref[...] * (gpre_ref[...] * inv)).astype(bf)
    bpre = bpre_ref[...].astype(bf)
    s1 = g1_ref[...] * inv
    h_dim = w1_ref.shape[1]
    w1a = (w1_ref[:h_dim, :] * s1).astype(bf)
    w1b = (w1_ref[h_dim:, :] * s1).astype(bf)
    w2 = (w2_ref[...] * (g2_ref[...] * inv)).astype(bf)
    b2 = b2_ref[...].astype(bf)
    xt = jnp.swapaxes(x3, 0, 1)
    hs = [(jnp.maximum(
              jnp.dot(xt[n], wpre, preferred_element_type=jnp.float32)
              .astype(bf) + bpre, 0) * m3[:, n:n + 1])
          for n in range(n_pts)]
    pooled = functools.reduce(jnp.maximum, hs)
    t2 = (jnp.dot(pooled, w1b, preferred_element_type=jnp.float32)
          + b1_ref[...]).astype(bf)
    poly = None
    for n in range(n_pts):
        zn = jnp.dot(hs[n], w1a, preferred_element_type=jnp.float32
                     ).astype(bf) + t2
        h2n = jnp.maximum(zn, 0)
        h2n = jnp.maximum(
            jnp.dot(h2n, w2, preferred_element_type=jnp.float32
                    ).astype(bf) + b2,
            0) * m3[:, n:n + 1]
        poly = h2n if poly is None else jnp.maximum(poly, h2n)
    valid = jnp.max(m3, axis=1, keepdims=True)
    out_ref[...] = (
        jnp.dot(poly.astype(jnp.float32), wout_ref[...],
                preferred_element_type=jnp.float32)
        + bout_ref[...]) * valid.astype(jnp.float32)


def kernel(polylines, polylines_mask, W_pre, g_pre, b_pre,
           W1, g1, b1, W2, g2, b2, W_out, b_out):
    B, P, N, C = polylines.shape
    H = W_pre.shape[1]
    O = W_out.shape[1]
    M = B * P
    tile = 768
    grid = M // tile

    xr = polylines.reshape(M, N, C)
    mr = polylines_mask.reshape(M, N)

    full = lambda shape: pl.BlockSpec(shape, lambda i: (0, 0))

    out = pl.pallas_call(
        functools.partial(_fused_encoder, n_pts=N, tile=tile),
        grid=(grid,),
        in_specs=[
            pl.BlockSpec((tile, N, C), lambda i: (i, 0, 0)),
            pl.BlockSpec((tile, N), lambda i: (i, 0)),
            full((C, H)),
            full((1, H)),
            full((1, H)),
            full((2 * H, H)),
            full((1, H)),
            full((1, H)),
            full((H, H)),
            full((1, H)),
            full((1, H)),
            full((H, O)),
            full((1, O)),
        ],
        out_specs=pl.BlockSpec((tile, O), lambda i: (i, 0)),
        out_shape=jax.ShapeDtypeStruct((M, O), jnp.float32),
        compiler_params=pltpu.CompilerParams(
            dimension_semantics=("parallel",)),
    )(xr, mr, W_pre, g_pre.reshape(1, H), b_pre.reshape(1, H),
      W1, g1.reshape(1, H), b1.reshape(1, H),
      W2, g2.reshape(1, H), b2.reshape(1, H),
      W_out, b_out.reshape(1, O))
    return out.reshape(B, P, O)


# speedup vs baseline: 1.0257x; 1.0257x over previous
import functools

import jax
import jax.numpy as jnp
from jax.experimental import pallas as pl
from jax.experimental.pallas import tpu as pltpu

_EPS = 1e-5


def _fused_encoder(x_ref, m_ref, wpre_ref, gpre_ref, bpre_ref,
                   w1_ref, g1_ref, b1_ref,
                   w2_ref, g2_ref, b2_ref,
                   wout_ref, bout_ref, out_ref, *, n_pts, tile):
    inv = 1.0 / jnp.sqrt(1.0 + _EPS)
    bf = jnp.bfloat16
    x3 = x_ref[...].astype(bf)
    m3 = m_ref[...].astype(bf)
    wpre = (wpre_ref[...] * (gpre_ref[...] * inv)).astype(bf)
    bpre = bpre_ref[...].astype(bf)
    s1 = g1_ref[...] * inv
    h_dim = w1_ref.shape[1]
    w1a = (w1_ref[:h_dim, :] * s1).astype(bf)
    w1b = (w1_ref[h_dim:, :] * s1).astype(bf)
    w2 = (w2_ref[...] * (g2_ref[...] * inv)).astype(bf)
    b2 = b2_ref[...].astype(bf)
    xt = jnp.swapaxes(x3, 0, 1)
    hs = [(jnp.maximum(
              jnp.dot(xt[n], wpre, preferred_element_type=jnp.float32)
              .astype(bf) + bpre, 0) * m3[:, n:n + 1])
          for n in range(n_pts)]
    def _treemax(vs):
        while len(vs) > 1:
            vs = [jnp.maximum(vs[i], vs[i + 1]) for i in range(0, len(vs) - 1, 2)] \
                 + ([vs[-1]] if len(vs) % 2 else [])
        return vs[0]

    pooled = _treemax(hs)
    t2 = (jnp.dot(pooled, w1b, preferred_element_type=jnp.float32)
          + b1_ref[...]).astype(bf)
    rows = n_pts * tile
    h = jnp.stack(hs, axis=0).reshape(rows, -1)
    z = jnp.dot(h, w1a, preferred_element_type=jnp.float32).astype(bf)
    z = (z.reshape(n_pts, tile, -1) + t2[None]).reshape(rows, -1)
    h2 = jnp.maximum(z, 0)
    h2 = jnp.maximum(
        jnp.dot(h2, w2, preferred_element_type=jnp.float32).astype(bf)
        + b2, 0)
    h2v = h2.reshape(n_pts, tile, -1)
    h2s = [h2v[n] * m3[:, n:n + 1] for n in range(n_pts)]
    poly = _treemax(h2s)
    valid = jnp.max(m3, axis=1, keepdims=True)
    out_ref[...] = (
        jnp.dot(poly.astype(jnp.float32), wout_ref[...],
                preferred_element_type=jnp.float32)
        + bout_ref[...]) * valid.astype(jnp.float32)


def kernel(polylines, polylines_mask, W_pre, g_pre, b_pre,
           W1, g1, b1, W2, g2, b2, W_out, b_out):
    B, P, N, C = polylines.shape
    H = W_pre.shape[1]
    O = W_out.shape[1]
    M = B * P
    tile = 512
    grid = M // tile

    xr = polylines.reshape(M, N, C)
    mr = polylines_mask.reshape(M, N)

    full = lambda shape: pl.BlockSpec(shape, lambda i: (0, 0))

    out = pl.pallas_call(
        functools.partial(_fused_encoder, n_pts=N, tile=tile),
        grid=(grid,),
        in_specs=[
            pl.BlockSpec((tile, N, C), lambda i: (i, 0, 0)),
            pl.BlockSpec((tile, N), lambda i: (i, 0)),
            full((C, H)),
            full((1, H)),
            full((1, H)),
            full((2 * H, H)),
            full((1, H)),
            full((1, H)),
            full((H, H)),
            full((1, H)),
            full((1, H)),
            full((H, O)),
            full((1, O)),
        ],
        out_specs=pl.BlockSpec((tile, O), lambda i: (i, 0)),
        out_shape=jax.ShapeDtypeStruct((M, O), jnp.float32),
        compiler_params=pltpu.CompilerParams(
            dimension_semantics=("parallel",)),
    )(xr, mr, W_pre, g_pre.reshape(1, H), b_pre.reshape(1, H),
      W1, g1.reshape(1, H), b1.reshape(1, H),
      W2, g2.reshape(1, H), b2.reshape(1, H),
      W_out, b_out.reshape(1, O))
    return out.reshape(B, P, O)


# speedup vs baseline: 1.0302x; 1.0045x over previous
import functools

import jax
import jax.numpy as jnp
from jax.experimental import pallas as pl
from jax.experimental.pallas import tpu as pltpu

_EPS = 1e-5


def _fused_encoder(x_ref, m_ref, wpre_ref, gpre_ref, bpre_ref,
                   w1_ref, g1_ref, b1_ref,
                   w2_ref, g2_ref, b2_ref,
                   wout_ref, bout_ref, out_ref, *, n_pts, tile):
    inv = 1.0 / jnp.sqrt(1.0 + _EPS)
    bf = jnp.bfloat16
    x3 = x_ref[...].astype(bf)
    m3 = m_ref[...].astype(bf)
    wpre = (wpre_ref[...] * (gpre_ref[...] * inv)).astype(bf)
    bpre = bpre_ref[...].astype(bf)
    s1 = g1_ref[...] * inv
    h_dim = w1_ref.shape[1]
    w1a = (w1_ref[:h_dim, :] * s1).astype(bf)
    w1b = (w1_ref[h_dim:, :] * s1).astype(bf)
    w2 = (w2_ref[...] * (g2_ref[...] * inv)).astype(bf)
    b2 = b2_ref[...].astype(bf)
    xt = jnp.swapaxes(x3, 0, 1)
    hs = [(jnp.maximum(
              jnp.dot(xt[n], wpre, preferred_element_type=jnp.float32)
              .astype(bf) + bpre, 0) * m3[:, n:n + 1])
          for n in range(n_pts)]
    def _treemax(vs):
        while len(vs) > 1:
            vs = [jnp.maximum(vs[i], vs[i + 1]) for i in range(0, len(vs) - 1, 2)] \
                 + ([vs[-1]] if len(vs) % 2 else [])
        return vs[0]

    pooled = _treemax(hs)
    t2 = (jnp.dot(pooled, w1b, preferred_element_type=jnp.float32)
          + b1_ref[...]).astype(bf)
    rows = n_pts * tile
    h = jnp.stack(hs, axis=0).reshape(rows, -1)
    z = jnp.dot(h, w1a, preferred_element_type=jnp.float32).astype(bf)
    z = (z.reshape(n_pts, tile, -1) + t2[None]).reshape(rows, -1)
    h2 = jnp.maximum(z, 0)
    h2 = jnp.maximum(
        jnp.dot(h2, w2, preferred_element_type=jnp.float32).astype(bf)
        + b2, 0)
    h2v = h2.reshape(n_pts, tile, -1)
    h2s = [h2v[n] * m3[:, n:n + 1] for n in range(n_pts)]
    poly = _treemax(h2s)
    valid = jnp.max(m3, axis=1, keepdims=True)
    out_ref[...] = (
        jnp.dot(poly, wout_ref[...].astype(bf),
                preferred_element_type=jnp.float32)
        + bout_ref[...]) * valid.astype(jnp.float32)


def kernel(polylines, polylines_mask, W_pre, g_pre, b_pre,
           W1, g1, b1, W2, g2, b2, W_out, b_out):
    B, P, N, C = polylines.shape
    H = W_pre.shape[1]
    O = W_out.shape[1]
    M = B * P
    tile = 512
    grid = M // tile

    xr = polylines.reshape(M, N, C)
    mr = polylines_mask.reshape(M, N)

    full = lambda shape: pl.BlockSpec(shape, lambda i: (0, 0))

    out = pl.pallas_call(
        functools.partial(_fused_encoder, n_pts=N, tile=tile),
        grid=(grid,),
        in_specs=[
            pl.BlockSpec((tile, N, C), lambda i: (i, 0, 0)),
            pl.BlockSpec((tile, N), lambda i: (i, 0)),
            full((C, H)),
            full((1, H)),
            full((1, H)),
            full((2 * H, H)),
            full((1, H)),
            full((1, H)),
            full((H, H)),
            full((1, H)),
            full((1, H)),
            full((H, O)),
            full((1, O)),
        ],
        out_specs=pl.BlockSpec((tile, O), lambda i: (i, 0)),
        out_shape=jax.ShapeDtypeStruct((M, O), jnp.float32),
        compiler_params=pltpu.CompilerParams(
            dimension_semantics=("parallel",)),
    )(xr, mr, W_pre, g_pre.reshape(1, H), b_pre.reshape(1, H),
      W1, g1.reshape(1, H), b1.reshape(1, H),
      W2, g2.reshape(1, H), b2.reshape(1, H),
      W_out, b_out.reshape(1, O))
    return out.reshape(B, P, O)


# speedup vs baseline: 1.0370x; 1.0066x over previous
import functools

import jax
import jax.numpy as jnp
from jax.experimental import pallas as pl
from jax.experimental.pallas import tpu as pltpu

_EPS = 1e-5


def _fused_encoder(x_ref, m_ref, wpre_ref, gpre_ref, bpre_ref,
                   w1_ref, g1_ref, b1_ref,
                   w2_ref, g2_ref, b2_ref,
                   wout_ref, bout_ref, out_ref, *, n_pts, tile):
    inv = 1.0 / jnp.sqrt(1.0 + _EPS)
    bf = jnp.bfloat16
    x3 = x_ref[...].astype(bf)
    m3 = m_ref[...].astype(bf)
    wpre = (wpre_ref[...] * (gpre_ref[...] * inv)).astype(bf)
    bpre = bpre_ref[...].astype(bf)
    s1 = g1_ref[...] * inv
    h_dim = w1_ref.shape[1]
    w1a = (w1_ref[:h_dim, :] * s1).astype(bf)
    w1b = (w1_ref[h_dim:, :] * s1).astype(bf)
    w2 = (w2_ref[...] * (g2_ref[...] * inv)).astype(bf)
    b2 = b2_ref[...].astype(bf)
    xt = jnp.swapaxes(x3, 0, 1)
    hs = [(jnp.maximum(
              jnp.dot(xt[n], wpre, preferred_element_type=jnp.float32)
              .astype(bf) + bpre, 0) * m3[:, n:n + 1])
          for n in range(n_pts)]
    def _treemax(vs):
        while len(vs) > 1:
            vs = [jnp.maximum(vs[i], vs[i + 1]) for i in range(0, len(vs) - 1, 2)] \
                 + ([vs[-1]] if len(vs) % 2 else [])
        return vs[0]

    pooled = _treemax(hs)
    t2 = (jnp.dot(pooled, w1b, preferred_element_type=jnp.float32)
          + b1_ref[...]).astype(bf)
    rows = n_pts * tile
    h = jnp.stack(hs, axis=0).reshape(rows, -1)
    z = jnp.dot(h, w1a, preferred_element_type=jnp.float32).astype(bf)
    z = (z.reshape(n_pts, tile, -1) + t2[None]).reshape(rows, -1)
    h2 = jnp.maximum(z, 0)
    h2 = jnp.maximum(
        jnp.dot(h2, w2, preferred_element_type=jnp.float32).astype(bf)
        + b2, 0)
    h2v = h2.reshape(n_pts, tile, -1)
    h2s = [h2v[n] * m3[:, n:n + 1] for n in range(n_pts)]
    poly = _treemax(h2s)
    valid = jnp.max(m3, axis=1, keepdims=True)
    out_ref[...] = (
        jnp.dot(poly, wout_ref[...].astype(bf),
                preferred_element_type=jnp.float32)
        + bout_ref[...]) * valid.astype(jnp.float32)


def kernel(polylines, polylines_mask, W_pre, g_pre, b_pre,
           W1, g1, b1, W2, g2, b2, W_out, b_out):
    B, P, N, C = polylines.shape
    H = W_pre.shape[1]
    O = W_out.shape[1]
    M = B * P
    tile = 384
    grid = M // tile

    xr = polylines.reshape(M, N, C)
    mr = polylines_mask.reshape(M, N)

    full = lambda shape: pl.BlockSpec(shape, lambda i: (0, 0))

    out = pl.pallas_call(
        functools.partial(_fused_encoder, n_pts=N, tile=tile),
        grid=(grid,),
        in_specs=[
            pl.BlockSpec((tile, N, C), lambda i: (i, 0, 0)),
            pl.BlockSpec((tile, N), lambda i: (i, 0)),
            full((C, H)),
            full((1, H)),
            full((1, H)),
            full((2 * H, H)),
            full((1, H)),
            full((1, H)),
            full((H, H)),
            full((1, H)),
            full((1, H)),
            full((H, O)),
            full((1, O)),
        ],
        out_specs=pl.BlockSpec((tile, O), lambda i: (i, 0)),
        out_shape=jax.ShapeDtypeStruct((M, O), jnp.float32),
        compiler_params=pltpu.CompilerParams(
            dimension_semantics=("parallel",)),
    )(xr, mr, W_pre, g_pre.reshape(1, H), b_pre.reshape(1, H),
      W1, g1.reshape(1, H), b1.reshape(1, H),
      W2, g2.reshape(1, H), b2.reshape(1, H),
      W_out, b_out.reshape(1, O))
    return out.reshape(B, P, O)
